# baseline (speedup 1.0000x reference)
import functools

import jax
import jax.numpy as jnp
from jax import lax
from jax.experimental import pallas as pl
from jax.experimental.pallas import tpu as pltpu
from jax.experimental.pallas import tpu_sc as plsc

_H = 128
_COLS = 256
_CPT = 8
_BIG = 1 << 20


def _best_kernel(radar_ref, mde_ref, enc_ref):
    H, C = radar_ref.shape
    mde = mde_ref[...]
    mde_valid = mde != 0.0
    has_mde = jnp.any(mde_valid, axis=0, keepdims=True)
    posi = lax.broadcasted_iota(jnp.int32, (H, C), 0)

    def step(y, carry):
        d_r = radar_ref[pl.ds(y, 1), :]
        diffs = jnp.where(mde_valid, jnp.abs(mde - d_r), jnp.inf)
        m = jnp.min(diffs, axis=0, keepdims=True)
        bidx = jnp.min(jnp.where(diffs == m, posi, H), axis=0, keepdims=True)
        best = jnp.where(has_mde, bidx, y)
        enc_ref[pl.ds(y, 1), :] = jnp.where(d_r != 0.0, best, -1)
        return carry

    lax.fori_loop(0, H, step, 0)


def _sload(ref, pos):
    idx = jnp.full((16,), 0, jnp.int32) + pos
    return plsc.load_gather(ref, [idx])[0]


def _sstore(ref, idxs, val, lane0):
    vecs = [jnp.full((16,), 0, jnp.int32) + i for i in idxs]
    plsc.store_scatter(ref, vecs, jnp.full((16,), 0, jnp.int32) + val,
                       mask=lane0)


def _find(ref, start, lane0):
    def cond(c):
        p, q = c
        return q != p

    def body(c):
        p, q = c
        r = _sload(ref, q)
        _sstore(ref, [p], r, lane0)
        return q, r

    p, _ = lax.while_loop(cond, body, (start, _sload(ref, start)))
    return p


def _sc_resolve_kernel(enc_hbm, vals_hbm, out_hbm, enc_v, vals_v, occ_v,
                       nxt_v, prv_v):
    wid = lax.axis_index("s") * 2 + lax.axis_index("c")
    base = wid * _CPT
    pltpu.sync_copy(enc_hbm.at[pl.ds(base, _CPT)], enc_v)
    pltpu.sync_copy(vals_hbm.at[pl.ds(base, _CPT)], vals_v)
    zero16 = jnp.zeros((16,), jnp.int32)
    lane0 = lax.broadcasted_iota(jnp.int32, (16,), 0) == 0

    for ci in range(_CPT):
        for k in range(10):
            i16 = lax.broadcasted_iota(jnp.int32, (16,), 0) + (16 * k)
            nxt_v[pl.ds(16 * k, 16)] = i16
            prv_v[pl.ds(16 * k, 16)] = i16
        for k in range(_H // 16):
            occ_v[ci, pl.ds(16 * k, 16)] = zero16

        def chunk(k, carry, ci=ci):
            enc_c = enc_v[ci, pl.ds(k * 16, 16)]
            val_c = vals_v[ci, pl.ds(k * 16, 16)]
            for j in range(16):
                b = enc_c[j]

                @pl.when(b >= 0)
                def _(b=b, j=j):
                    fu = _find(nxt_v, b, lane0)
                    fdi = _find(prv_v, b + 1, lane0)
                    du = jnp.where(fu < _H, fu - b, _BIG)
                    dd = jnp.where(fdi > 0, b + 1 - fdi, _BIG)
                    final = jnp.where(du <= dd, fu, fdi - 1)
                    final = jnp.where(jnp.minimum(du, dd) >= _BIG, b, final)
                    _sstore(occ_v, [ci, final], val_c[j], lane0)
                    _sstore(nxt_v, [final], final + 1, lane0)
                    _sstore(prv_v, [final + 1], final, lane0)

            return carry

        lax.fori_loop(0, _H // 16, chunk, 0)

    pltpu.sync_copy(occ_v, out_hbm.at[pl.ds(base, _CPT)])


@functools.partial(
    pl.kernel,
    out_type=jax.ShapeDtypeStruct((_COLS, _H), jnp.int32),
    mesh=plsc.VectorSubcoreMesh(core_axis_name="c", subcore_axis_name="s"),
    compiler_params=pltpu.CompilerParams(needs_layout_passes=False),
    scratch_types=[
        pltpu.VMEM((_CPT, _H), jnp.int32),
        pltpu.VMEM((_CPT, _H), jnp.int32),
        pltpu.VMEM((_CPT, _H), jnp.int32),
        pltpu.VMEM((160,), jnp.int32),
        pltpu.VMEM((160,), jnp.int32),
    ],
)
def _sc_resolve(enc_hbm, vals_hbm, out_hbm, enc_v, vals_v, occ_v, nxt_v,
                prv_v):
    _sc_resolve_kernel(enc_hbm, vals_hbm, out_hbm, enc_v, vals_v, occ_v,
                       nxt_v, prv_v)


def kernel(radar_patches, mde_out_patches):
    W, B, C, H, _ = radar_patches.shape
    radar_cols = radar_patches[:, :, 0, :, 0].reshape(W * B, H)
    mde_cols = mde_out_patches[:, :, 0, :, 0].reshape(W * B, H)

    enc_t = pl.pallas_call(
        _best_kernel,
        out_shape=jax.ShapeDtypeStruct((H, W * B), jnp.int32),
    )(radar_cols.T, mde_cols.T)

    enc_cols = enc_t.T
    vals_bits = lax.bitcast_convert_type(radar_cols, jnp.int32)
    occ_bits = _sc_resolve(enc_cols, vals_bits)
    occ = lax.bitcast_convert_type(occ_bits, jnp.float32)

    cols = occ.reshape(W, B, H)
    radar_gt = jnp.zeros((B, C, H, W), dtype=jnp.float32)
    radar_gt = radar_gt.at[:, 0, :, :].set(jnp.transpose(cols, (1, 2, 0)))
    return radar_gt


# speedup vs baseline: 1.9592x; 1.9592x over previous
import functools

import jax
import jax.numpy as jnp
from jax import lax
from jax.experimental import pallas as pl
from jax.experimental.pallas import tpu as pltpu
from jax.experimental.pallas import tpu_sc as plsc

_H = 128
_COLS = 256
_CPT = 8
_BIG = 1 << 20


def _best_kernel(radar_ref, mde_ref, enc_ref):
    H, C = radar_ref.shape
    mde = mde_ref[...]
    mde_valid = mde != 0.0
    has_mde = jnp.any(mde_valid, axis=0, keepdims=True)
    posi = lax.broadcasted_iota(jnp.int32, (H, C), 0)

    def step(y, carry):
        d_r = radar_ref[pl.ds(y, 1), :]
        diffs = jnp.where(mde_valid, jnp.abs(mde - d_r), jnp.inf)
        m = jnp.min(diffs, axis=0, keepdims=True)
        bidx = jnp.min(jnp.where(diffs == m, posi, H), axis=0, keepdims=True)
        best = jnp.where(has_mde, bidx, y)
        enc_ref[pl.ds(y, 1), :] = jnp.where(d_r != 0.0, best, -1)
        return carry

    lax.fori_loop(0, H, step, 0)


def _sstore(ref, idxs, val, lane0):
    vecs = [jnp.full((16,), 0, jnp.int32) + i for i in idxs]
    plsc.store_scatter(ref, vecs, jnp.full((16,), 0, jnp.int32) + val,
                       mask=lane0)


def _find(ref, start, first):
    def cond(c):
        p, q = c
        return q != p

    def body(c):
        p, q = c
        r = ref[q]
        ref[p] = r
        return q, r

    p, _ = lax.while_loop(cond, body, (start, first))
    return p


def _sc_resolve_kernel(enc_hbm, vals_hbm, out_hbm, enc_v, vals_v, occ_v,
                       nxt_s, prv_s):
    wid = lax.axis_index("s") * 2 + lax.axis_index("c")
    base = wid * _CPT
    pltpu.sync_copy(enc_hbm.at[pl.ds(base, _CPT)], enc_v)
    pltpu.sync_copy(vals_hbm.at[pl.ds(base, _CPT)], vals_v)
    zero16 = jnp.zeros((16,), jnp.int32)
    lane0 = lax.broadcasted_iota(jnp.int32, (16,), 0) == 0

    for ci in range(_CPT):
        def init(i, carry):
            for u in range(8):
                nxt_s[i * 8 + u] = i * 8 + u
                prv_s[i * 8 + u] = i * 8 + u
            return carry

        lax.fori_loop(0, 20, init, 0)
        for k in range(_H // 16):
            occ_v[ci, pl.ds(16 * k, 16)] = zero16

        def chunk(k, carry, ci=ci):
            enc_c = enc_v[ci, pl.ds(k * 16, 16)]
            val_c = vals_v[ci, pl.ds(k * 16, 16)]
            for j in range(16):
                b = enc_c[j]

                @pl.when(b >= 0)
                def _(b=b, j=j):
                    nb = nxt_s[b]

                    def fast(_):
                        return b

                    def slow(_):
                        fu = _find(nxt_s, b, nb)
                        i0 = b + 1
                        fdi = _find(prv_s, i0, prv_s[i0])
                        du = jnp.where(fu < _H, fu - b, _BIG)
                        dd = jnp.where(fdi > 0, i0 - fdi, _BIG)
                        f = jnp.where(du <= dd, fu, fdi - 1)
                        return jnp.where(jnp.minimum(du, dd) >= _BIG, b, f)

                    final = lax.cond(nb == b, fast, slow, 0)
                    _sstore(occ_v, [ci, final], val_c[j], lane0)
                    nxt_s[final] = final + 1
                    prv_s[final + 1] = final

            return carry

        lax.fori_loop(0, _H // 16, chunk, 0)

    pltpu.sync_copy(occ_v, out_hbm.at[pl.ds(base, _CPT)])


@functools.partial(
    pl.kernel,
    out_type=jax.ShapeDtypeStruct((_COLS, _H), jnp.int32),
    mesh=plsc.VectorSubcoreMesh(core_axis_name="c", subcore_axis_name="s"),
    compiler_params=pltpu.CompilerParams(needs_layout_passes=False),
    scratch_types=[
        pltpu.VMEM((_CPT, _H), jnp.int32),
        pltpu.VMEM((_CPT, _H), jnp.int32),
        pltpu.VMEM((_CPT, _H), jnp.int32),
        pltpu.SMEM((160,), jnp.int32),
        pltpu.SMEM((160,), jnp.int32),
    ],
)
def _sc_resolve(enc_hbm, vals_hbm, out_hbm, enc_v, vals_v, occ_v, nxt_v,
                prv_v):
    _sc_resolve_kernel(enc_hbm, vals_hbm, out_hbm, enc_v, vals_v, occ_v,
                       nxt_v, prv_v)


def kernel(radar_patches, mde_out_patches):
    W, B, C, H, _ = radar_patches.shape
    radar_cols = radar_patches[:, :, 0, :, 0].reshape(W * B, H)
    mde_cols = mde_out_patches[:, :, 0, :, 0].reshape(W * B, H)

    enc_t = pl.pallas_call(
        _best_kernel,
        out_shape=jax.ShapeDtypeStruct((H, W * B), jnp.int32),
    )(radar_cols.T, mde_cols.T)

    enc_cols = enc_t.T
    vals_bits = lax.bitcast_convert_type(radar_cols, jnp.int32)
    occ_bits = _sc_resolve(enc_cols, vals_bits)
    occ = lax.bitcast_convert_type(occ_bits, jnp.float32)

    cols = occ.reshape(W, B, H)
    radar_gt = jnp.zeros((B, C, H, W), dtype=jnp.float32)
    radar_gt = radar_gt.at[:, 0, :, :].set(jnp.transpose(cols, (1, 2, 0)))
    return radar_gt


# speedup vs baseline: 2.3026x; 1.1753x over previous
import functools

import jax
import jax.numpy as jnp
from jax import lax
from jax.experimental import pallas as pl
from jax.experimental.pallas import tpu as pltpu
from jax.experimental.pallas import tpu_sc as plsc

_H = 128
_COLS = 256
_CPT = 8
_BIG = 1 << 20


def _best_kernel(radar_ref, mde_ref, enc_ref):
    H, C = radar_ref.shape
    mde = mde_ref[...]
    mde_valid = mde != 0.0
    has_mde = jnp.any(mde_valid, axis=0, keepdims=True)
    posi = lax.broadcasted_iota(jnp.int32, (H, C), 0)

    def step(y, carry):
        d_r = radar_ref[pl.ds(y, 1), :]
        diffs = jnp.where(mde_valid, jnp.abs(mde - d_r), jnp.inf)
        m = jnp.min(diffs, axis=0, keepdims=True)
        bidx = jnp.min(jnp.where(diffs == m, posi, H), axis=0, keepdims=True)
        best = jnp.where(has_mde, bidx, y)
        enc_ref[pl.ds(y, 1), :] = jnp.where(d_r != 0.0, best, -1)
        return carry

    lax.fori_loop(0, H, step, 0)


def _sstore(ref, idxs, val, lane0):
    vecs = [jnp.full((16,), 0, jnp.int32) + i for i in idxs]
    plsc.store_scatter(ref, vecs, jnp.full((16,), 0, jnp.int32) + val,
                       mask=lane0)


def _find(ref, start, first):
    def cond(c):
        p, q = c
        return q != p

    def body(c):
        p, q = c
        r = ref[q]
        ref[p] = r
        return q, r

    p, _ = lax.while_loop(cond, body, (start, first))
    return p


def _sc_resolve_kernel(enc_hbm, vals_hbm, out_hbm, enc_v, vals_v, occ_v,
                       perm_v, nxt_s, fnx_s, fpv_s):
    wid = lax.axis_index("s") * 2 + lax.axis_index("c")
    base = wid * _CPT
    pltpu.sync_copy(enc_hbm.at[pl.ds(base, _CPT)], enc_v)
    pltpu.sync_copy(vals_hbm.at[pl.ds(base, _CPT)], vals_v)
    lane0 = lax.broadcasted_iota(jnp.int32, (16,), 0) == 0
    lanes = lax.broadcasted_iota(jnp.int32, (16,), 0)
    big16 = jnp.full((16,), 2 * _H, jnp.int32)

    for ci in range(_CPT):
        ci_vec = jnp.full((16,), ci, jnp.int32)
        def init(i, carry):
            for u in range(4):
                t = i * 4 + u
                nxt_s[t] = t
                fnx_s[t] = t + 1
                fpv_s[t] = t - 1
            return carry

        lax.fori_loop(0, (_H + 4) // 4, init, 0)
        fnx_s[_H] = 0
        fpv_s[0] = _H
        for k in range(_H // 16):
            perm_v[ci, pl.ds(16 * k, 16)] = big16

        def chunk(k, carry, ci=ci):
            enc_c = enc_v[ci, pl.ds(k * 16, 16)]
            for j in range(16):
                b = enc_c[j]

                @pl.when(b >= 0)
                def _(b=b, j=j, k=k):
                    fu = _find(nxt_s, b, nxt_s[b])
                    fd = fpv_s[fu]
                    du = jnp.where(fu < _H, fu - b, _BIG)
                    dd = jnp.where(fd < _H, b - fd, _BIG)
                    final = jnp.where(du <= dd, fu, fd)
                    final = jnp.where(jnp.minimum(du, dd) >= _BIG, b, final)
                    _sstore(perm_v, [ci, final], k * 16 + j, lane0)
                    nxt_s[final] = final + 1
                    a = fpv_s[final]
                    c = fnx_s[final]
                    fnx_s[a] = c
                    fpv_s[c] = a

            return carry

        lax.fori_loop(0, _H // 16, chunk, 0)

        for k in range(_H // 16):
            idx = perm_v[ci, pl.ds(16 * k, 16)]
            got = plsc.load_gather(vals_v, [ci_vec, jnp.minimum(idx, _H - 1)])
            occ_v[ci, pl.ds(16 * k, 16)] = jnp.where(idx < _H, got, 0)

    pltpu.sync_copy(occ_v, out_hbm.at[pl.ds(base, _CPT)])


@functools.partial(
    pl.kernel,
    out_type=jax.ShapeDtypeStruct((_COLS, _H), jnp.int32),
    mesh=plsc.VectorSubcoreMesh(core_axis_name="c", subcore_axis_name="s"),
    compiler_params=pltpu.CompilerParams(needs_layout_passes=False),
    scratch_types=[
        pltpu.VMEM((_CPT, _H), jnp.int32),
        pltpu.VMEM((_CPT, _H), jnp.int32),
        pltpu.VMEM((_CPT, _H), jnp.int32),
        pltpu.VMEM((_CPT, _H), jnp.int32),
        pltpu.SMEM((160,), jnp.int32),
        pltpu.SMEM((160,), jnp.int32),
        pltpu.SMEM((160,), jnp.int32),
    ],
)
def _sc_resolve(enc_hbm, vals_hbm, out_hbm, enc_v, vals_v, occ_v, perm_v,
                nxt_s, fnx_s, fpv_s):
    _sc_resolve_kernel(enc_hbm, vals_hbm, out_hbm, enc_v, vals_v, occ_v,
                       perm_v, nxt_s, fnx_s, fpv_s)


def kernel(radar_patches, mde_out_patches):
    W, B, C, H, _ = radar_patches.shape
    radar_cols = radar_patches[:, :, 0, :, 0].reshape(W * B, H)
    mde_cols = mde_out_patches[:, :, 0, :, 0].reshape(W * B, H)

    enc_t = pl.pallas_call(
        _best_kernel,
        out_shape=jax.ShapeDtypeStruct((H, W * B), jnp.int32),
    )(radar_cols.T, mde_cols.T)

    enc_cols = enc_t.T
    vals_bits = lax.bitcast_convert_type(radar_cols, jnp.int32)
    occ_bits = _sc_resolve(enc_cols, vals_bits)
    occ = lax.bitcast_convert_type(occ_bits, jnp.float32)

    cols = occ.reshape(W, B, H)
    radar_gt = jnp.zeros((B, C, H, W), dtype=jnp.float32)
    radar_gt = radar_gt.at[:, 0, :, :].set(jnp.transpose(cols, (1, 2, 0)))
    return radar_gt


# speedup vs baseline: 4.7669x; 2.0702x over previous
import functools

import jax
import jax.numpy as jnp
from jax import lax
from jax.experimental import pallas as pl
from jax.experimental.pallas import tpu as pltpu
from jax.experimental.pallas import tpu_sc as plsc

_H = 128
_COLS = 256
_LPT = 16
_NT = _COLS // _LPT
_BIG = 1 << 20


def _best_kernel(radar_ref, mde_ref, enc_ref):
    H, C = radar_ref.shape
    mde = mde_ref[...]
    mde_valid = mde != 0.0
    has_mde = jnp.any(mde_valid, axis=0, keepdims=True)
    posi = lax.broadcasted_iota(jnp.int32, (H, C), 0)

    def step(y, carry):
        d_r = radar_ref[pl.ds(y, 1), :]
        diffs = jnp.where(mde_valid, jnp.abs(mde - d_r), jnp.inf)
        m = jnp.min(diffs, axis=0, keepdims=True)
        bidx = jnp.min(jnp.where(diffs == m, posi, H), axis=0, keepdims=True)
        best = jnp.where(has_mde, bidx, y)
        enc_ref[pl.ds(y, 1), :] = jnp.where(d_r != 0.0, best, -1)
        return carry

    lax.fori_loop(0, H, step, 0)


def _exponent(v):
    f = v.astype(jnp.float32)
    return (lax.bitcast_convert_type(f, jnp.int32) >> 23) - 127


def _sc_resolve_kernel(enc_hbm, vals_hbm, out_hbm, enc_v, vals_v, occ_v):
    wid = lax.axis_index("s") * 2 + lax.axis_index("c")

    @pl.when(wid < _NT)
    def _():
        base = wid * _LPT
        pltpu.sync_copy(enc_hbm.at[pl.ds(base, _LPT)], enc_v)
        pltpu.sync_copy(vals_hbm.at[pl.ds(base, _LPT)], vals_v)
        lanes = lax.broadcasted_iota(jnp.int32, (16,), 0)
        zero16 = jnp.zeros((16,), jnp.int32)
        ones = jnp.full((16,), 1, jnp.int32)
        full = jnp.full((16,), 0xFFFF, jnp.int32)
        for c in range(_LPT):
            for k in range(_H // 16):
                occ_v[c, pl.ds(16 * k, 16)] = zero16

        def step(y, fw):
            y16 = zero16 + y
            b = plsc.load_gather(enc_v, [lanes, y16])
            vals = plsc.load_gather(vals_v, [lanes, y16])
            wb = b >> 4
            rb = b & 15
            himask = (full << rb) & full
            lomask = (ones << (rb + 1)) - 1
            vu, wu = ones, jnp.full((16,), 8, jnp.int32)
            for i in range(7, -1, -1):
                sel = jnp.where(wb < i, full, jnp.where(wb == i, himask, 0))
                mi = fw[i] & sel
                nz = mi != 0
                vu = jnp.where(nz, mi, vu)
                wu = jnp.where(nz, i, wu)
            fu = wu * 16 + _exponent(vu & (-vu))
            vd, wd = ones, jnp.full((16,), -8, jnp.int32)
            for i in range(8):
                sel = jnp.where(wb > i, full, jnp.where(wb == i, lomask, 0))
                mi = fw[i] & sel
                nz = mi != 0
                vd = jnp.where(nz, mi, vd)
                wd = jnp.where(nz, i, wd)
            fd = wd * 16 + _exponent(vd)
            du = jnp.where(fu < _H, fu - b, _BIG)
            dd = jnp.where(fd >= 0, b - fd, _BIG)
            final = jnp.where(du <= dd, fu, fd)
            final = jnp.clip(final, 0, _H - 1)
            write = b >= 0
            plsc.store_scatter(occ_v, [lanes, final], vals, mask=write)
            wf = final >> 4
            clearbit = jnp.where(write, ones << (final & 15), zero16)
            return tuple(
                jnp.where(wf == i, fw[i] & ~clearbit, fw[i]) for i in range(8)
            )

        lax.fori_loop(0, _H, step, (full,) * 8)
        pltpu.sync_copy(occ_v, out_hbm.at[pl.ds(base, _LPT)])


@functools.partial(
    pl.kernel,
    out_type=jax.ShapeDtypeStruct((_COLS, _H), jnp.int32),
    mesh=plsc.VectorSubcoreMesh(core_axis_name="c", subcore_axis_name="s"),
    compiler_params=pltpu.CompilerParams(needs_layout_passes=False),
    scratch_types=[
        pltpu.VMEM((_LPT, _H), jnp.int32),
        pltpu.VMEM((_LPT, _H), jnp.int32),
        pltpu.VMEM((_LPT, _H), jnp.int32),
    ],
)
def _sc_resolve(enc_hbm, vals_hbm, out_hbm, enc_v, vals_v, occ_v):
    _sc_resolve_kernel(enc_hbm, vals_hbm, out_hbm, enc_v, vals_v, occ_v)


def kernel(radar_patches, mde_out_patches):
    W, B, C, H, _ = radar_patches.shape
    radar_cols = radar_patches[:, :, 0, :, 0].reshape(W * B, H)
    mde_cols = mde_out_patches[:, :, 0, :, 0].reshape(W * B, H)
    radar_t = radar_cols.T
    mde_t = mde_cols.T

    enc_t = pl.pallas_call(
        _best_kernel,
        out_shape=jax.ShapeDtypeStruct((H, W * B), jnp.int32),
    )(radar_t, mde_t)

    enc_cols = enc_t.T
    vals_bits = lax.bitcast_convert_type(radar_cols, jnp.int32)
    occ_bits = _sc_resolve(enc_cols, vals_bits)
    occ = lax.bitcast_convert_type(occ_bits, jnp.float32)

    cols = occ.reshape(W, B, H)
    radar_gt = jnp.zeros((B, C, H, W), dtype=jnp.float32)
    radar_gt = radar_gt.at[:, 0, :, :].set(jnp.transpose(cols, (1, 2, 0)))
    return radar_gt


# speedup vs baseline: 5.7187x; 1.1997x over previous
import functools

import jax
import jax.numpy as jnp
from jax import lax
from jax.experimental import pallas as pl
from jax.experimental.pallas import tpu as pltpu
from jax.experimental.pallas import tpu_sc as plsc

_H = 128
_COLS = 256
_LPT = 16
_NT = _COLS // _LPT
_NW = _H // 32
_BIG = 1 << 20


def _best_kernel(radar_ref, mde_ref, enc_ref, rt, mt, et):
    C, H = radar_ref.shape
    rt[...] = radar_ref[...].T
    mt[...] = mde_ref[...].T
    mde = mt[...]
    mde_valid = mde != 0.0
    has_mde = jnp.any(mde_valid, axis=0, keepdims=True)
    posi = lax.broadcasted_iota(jnp.int32, (H, C), 0)

    def step(y, carry):
        d_r = rt[pl.ds(y, 1), :]
        diffs = jnp.where(mde_valid, jnp.abs(mde - d_r), jnp.inf)
        m = jnp.min(diffs, axis=0, keepdims=True)
        bidx = jnp.min(jnp.where(diffs == m, posi, H), axis=0, keepdims=True)
        best = jnp.where(has_mde, bidx, y)
        et[pl.ds(y, 1), :] = jnp.where(d_r != 0.0, best, -1)
        return carry

    lax.fori_loop(0, H, step, 0)
    enc_ref[...] = et[...].T


def _lsb_exp(t):
    f = t.astype(jnp.float32)
    return ((lax.bitcast_convert_type(f, jnp.int32) >> 23) & 0xFF) - 127


def _sc_resolve_kernel(enc_hbm, vals_hbm, out_hbm, enc_v, vals_v, occ_v):
    wid = lax.axis_index("s") * 2 + lax.axis_index("c")

    @pl.when(wid < _NT)
    def _():
        base = wid * _LPT
        pltpu.sync_copy(enc_hbm.at[pl.ds(base, _LPT)], enc_v)
        pltpu.sync_copy(vals_hbm.at[pl.ds(base, _LPT)], vals_v)
        lanes = lax.broadcasted_iota(jnp.int32, (16,), 0)
        zero16 = jnp.zeros((16,), jnp.int32)
        ones = jnp.full((16,), 1, jnp.int32)
        full = jnp.full((16,), -1, jnp.int32)
        fzero = jnp.zeros((16,), jnp.float32)
        for c in range(_LPT):
            for k in range(_H // 16):
                occ_v[c, pl.ds(16 * k, 16)] = fzero

        def lsr(x, k):
            return lax.shift_right_logical(x, jnp.int32(k))

        def step(y, fw):
            y16 = zero16 + y
            b = plsc.load_gather(enc_v, [lanes, y16])
            vals = plsc.load_gather(vals_v, [lanes, y16])
            wb = b >> 5
            rb = b & 31
            hb = ones << rb
            himask = 0 - hb
            lomask = hb | (hb - 1)
            vu, wu = ones, jnp.full((16,), _NW, jnp.int32)
            for i in range(_NW - 1, -1, -1):
                sel = jnp.where(wb < i, full, jnp.where(wb == i, himask, 0))
                mi = fw[i] & sel
                nz = mi != 0
                vu = jnp.where(nz, mi, vu)
                wu = jnp.where(nz, i, wu)
            fu = wu * 32 + _lsb_exp(vu & (0 - vu))
            vd, wd = ones, jnp.full((16,), -_NW, jnp.int32)
            for i in range(_NW):
                sel = jnp.where(wb > i, full, jnp.where(wb == i, lomask, 0))
                mi = fw[i] & sel
                nz = mi != 0
                vd = jnp.where(nz, mi, vd)
                wd = jnp.where(nz, i, wd)
            s = vd | lsr(vd, 1)
            s = s | lsr(s, 2)
            s = s | lsr(s, 4)
            s = s | lsr(s, 8)
            s = s | lsr(s, 16)
            fd = wd * 32 + _lsb_exp(s ^ lsr(s, 1))
            du = jnp.where(fu < _H, fu - b, _BIG)
            dd = jnp.where(fd >= 0, b - fd, _BIG)
            final = jnp.clip(jnp.where(du <= dd, fu, fd), 0, _H - 1)
            write = b >= 0
            plsc.store_scatter(occ_v, [lanes, final], vals, mask=write)
            wf = final >> 5
            clearbit = jnp.where(write, ones << (final & 31), zero16)
            return tuple(
                jnp.where(wf == i, fw[i] & ~clearbit, fw[i])
                for i in range(_NW)
            )

        lax.fori_loop(0, _H, step, (full,) * _NW)
        pltpu.sync_copy(occ_v, out_hbm.at[pl.ds(base, _LPT)])


@functools.lru_cache(maxsize=None)
def _sc_resolve():
    return pl.kernel(
        _sc_resolve_kernel,
        out_type=jax.ShapeDtypeStruct((_COLS, _H), jnp.float32),
        mesh=plsc.VectorSubcoreMesh(core_axis_name="c", subcore_axis_name="s"),
        compiler_params=pltpu.CompilerParams(needs_layout_passes=False),
        scratch_types=[
            pltpu.VMEM((_LPT, _H), jnp.int32),
            pltpu.VMEM((_LPT, _H), jnp.float32),
            pltpu.VMEM((_LPT, _H), jnp.float32),
        ],
    )


def kernel(radar_patches, mde_out_patches):
    W, B, C, H, _ = radar_patches.shape
    radar_cols = radar_patches[:, :, 0, :, 0].reshape(W * B, H)
    mde_cols = mde_out_patches[:, :, 0, :, 0].reshape(W * B, H)

    enc = pl.pallas_call(
        _best_kernel,
        out_shape=jax.ShapeDtypeStruct((W * B, H), jnp.int32),
        scratch_shapes=[
            pltpu.VMEM((H, W * B), jnp.float32),
            pltpu.VMEM((H, W * B), jnp.float32),
            pltpu.VMEM((H, W * B), jnp.int32),
        ],
    )(radar_cols, mde_cols)

    occ = _sc_resolve()(enc, radar_cols)

    cols_t = jnp.transpose(occ.reshape(W, B, H), (1, 2, 0))
    if C == 1:
        return cols_t[:, None, :, :]
    radar_gt = jnp.zeros((B, C, H, W), dtype=jnp.float32)
    return radar_gt.at[:, 0, :, :].set(cols_t)


# speedup vs baseline: 6.0323x; 1.0548x over previous
import functools

import jax
import jax.numpy as jnp
from jax import lax
from jax.experimental import pallas as pl
from jax.experimental.pallas import tpu as pltpu
from jax.experimental.pallas import tpu_sc as plsc

_H = 128
_COLS = 256
_LPT = 16
_NT = _COLS // _LPT
_NW = _H // 32
_BIG = 1 << 20


_HUGE = jnp.inf


def _best_kernel(radar_ref, mde_ref, enc_ref, rt, mt, et):
    C, H = radar_ref.shape
    unroll = 4
    rt[...] = radar_ref[...].T
    mde_in = mde_ref[...]
    mt[...] = jnp.where(mde_in != 0.0, mde_in, _HUGE).T
    mdem = mt[...]
    has_mde = jnp.any(mdem != _HUGE, axis=0, keepdims=True)
    posi = lax.broadcasted_iota(jnp.int32, (H, C), 0)

    def step(i, carry):
        for k in range(unroll):
            y = i * unroll + k
            d_r = rt[pl.ds(y, 1), :]
            diffs = jnp.abs(mdem - d_r)
            m = jnp.min(diffs, axis=0, keepdims=True)
            bidx = jnp.min(
                jnp.where(diffs == m, posi, H), axis=0, keepdims=True
            )
            best = jnp.where(has_mde, bidx, y)
            et[pl.ds(y, 1), :] = jnp.where(d_r != 0.0, best, -1)
        return carry

    lax.fori_loop(0, H // unroll, step, 0)
    enc_ref[...] = et[...].T


def _lsb_exp(t):
    f = t.astype(jnp.float32)
    return ((lax.bitcast_convert_type(f, jnp.int32) >> 23) & 0xFF) - 127


def _sc_resolve_kernel(enc_hbm, vals_hbm, out_hbm, enc_v, vals_v, occ_v):
    wid = lax.axis_index("s") * 2 + lax.axis_index("c")

    @pl.when(wid < _NT)
    def _():
        base = wid * _LPT
        pltpu.sync_copy(enc_hbm.at[pl.ds(base, _LPT)], enc_v)
        pltpu.sync_copy(vals_hbm.at[pl.ds(base, _LPT)], vals_v)
        lanes = lax.broadcasted_iota(jnp.int32, (16,), 0)
        zero16 = jnp.zeros((16,), jnp.int32)
        ones = jnp.full((16,), 1, jnp.int32)
        full = jnp.full((16,), -1, jnp.int32)
        fzero = jnp.zeros((16,), jnp.float32)
        for c in range(_LPT):
            for k in range(_H // 16):
                occ_v[c, pl.ds(16 * k, 16)] = fzero

        def lsr(x, k):
            return lax.shift_right_logical(x, jnp.int32(k))

        def step(y, fw):
            y16 = zero16 + y
            b = plsc.load_gather(enc_v, [lanes, y16])
            vals = plsc.load_gather(vals_v, [lanes, y16])
            wb = b >> 5
            rb = b & 31
            hb = ones << rb
            himask = 0 - hb
            lomask = hb | (hb - 1)
            vu, wu = ones, jnp.full((16,), _NW, jnp.int32)
            for i in range(_NW - 1, -1, -1):
                sel = jnp.where(wb < i, full, jnp.where(wb == i, himask, 0))
                mi = fw[i] & sel
                nz = mi != 0
                vu = jnp.where(nz, mi, vu)
                wu = jnp.where(nz, i, wu)
            fu = wu * 32 + _lsb_exp(vu & (0 - vu))
            vd, wd = ones, jnp.full((16,), -_NW, jnp.int32)
            for i in range(_NW):
                sel = jnp.where(wb > i, full, jnp.where(wb == i, lomask, 0))
                mi = fw[i] & sel
                nz = mi != 0
                vd = jnp.where(nz, mi, vd)
                wd = jnp.where(nz, i, wd)
            s = vd | lsr(vd, 1)
            s = s | lsr(s, 2)
            s = s | lsr(s, 4)
            s = s | lsr(s, 8)
            s = s | lsr(s, 16)
            fd = wd * 32 + _lsb_exp(s ^ lsr(s, 1))
            du = jnp.where(fu < _H, fu - b, _BIG)
            dd = jnp.where(fd >= 0, b - fd, _BIG)
            final = jnp.clip(jnp.where(du <= dd, fu, fd), 0, _H - 1)
            write = b >= 0
            plsc.store_scatter(occ_v, [lanes, final], vals, mask=write)
            wf = final >> 5
            clearbit = jnp.where(write, ones << (final & 31), zero16)
            return tuple(
                jnp.where(wf == i, fw[i] & ~clearbit, fw[i])
                for i in range(_NW)
            )

        lax.fori_loop(0, _H, step, (full,) * _NW)
        pltpu.sync_copy(occ_v, out_hbm.at[pl.ds(base, _LPT)])


@functools.lru_cache(maxsize=None)
def _sc_resolve():
    return pl.kernel(
        _sc_resolve_kernel,
        out_type=jax.ShapeDtypeStruct((_COLS, _H), jnp.float32),
        mesh=plsc.VectorSubcoreMesh(core_axis_name="c", subcore_axis_name="s"),
        compiler_params=pltpu.CompilerParams(needs_layout_passes=False),
        scratch_types=[
            pltpu.VMEM((_LPT, _H), jnp.int32),
            pltpu.VMEM((_LPT, _H), jnp.float32),
            pltpu.VMEM((_LPT, _H), jnp.float32),
        ],
    )


def kernel(radar_patches, mde_out_patches):
    W, B, C, H, _ = radar_patches.shape
    radar_cols = radar_patches[:, :, 0, :, 0].reshape(W * B, H)
    mde_cols = mde_out_patches[:, :, 0, :, 0].reshape(W * B, H)

    enc = pl.pallas_call(
        _best_kernel,
        out_shape=jax.ShapeDtypeStruct((W * B, H), jnp.int32),
        scratch_shapes=[
            pltpu.VMEM((H, W * B), jnp.float32),
            pltpu.VMEM((H, W * B), jnp.float32),
            pltpu.VMEM((H, W * B), jnp.int32),
        ],
    )(radar_cols, mde_cols)

    occ = _sc_resolve()(enc, radar_cols)

    cols_t = jnp.transpose(occ.reshape(W, B, H), (1, 2, 0))
    if C == 1:
        return cols_t[:, None, :, :]
    radar_gt = jnp.zeros((B, C, H, W), dtype=jnp.float32)
    return radar_gt.at[:, 0, :, :].set(cols_t)
